# Initial kernel scaffold; baseline (speedup 1.0000x reference)
import jax
import jax.numpy as jnp
from jax.experimental import pallas as pl


def kernel(mesh_pos, edges, state, node_type, parameters, params):
    raise NotImplementedError("write your pallas kernel here")


# speedup vs baseline: 1595.9959x; 1595.9959x over previous
import functools

import jax
import jax.numpy as jnp
from jax import lax
from jax.experimental import pallas as pl
from jax.experimental.pallas import tpu as pltpu
from jax.experimental.pallas import tpu_sc as plsc

N = 10000
E = 40000
D = 128
NPAD = 10240
EPAD = 40960
IPAD = 2 * EPAD
BR = 512
GW = 128

_f32 = jnp.float32


def _sds(shape):
    return jax.ShapeDtypeStruct(shape, _f32)


@functools.lru_cache
def _sc_gather(vdim):
    mesh = plsc.VectorSubcoreMesh(
        core_axis_name="core", subcore_axis_name="subcore",
        num_cores=2, num_subcores=16)

    @functools.partial(
        pl.kernel,
        out_type=_sds((IPAD, vdim)),
        mesh=mesh,
    )
    def gather_k(x_hbm, i_hbm, o_hbm):
        def body(i_vmem, o_vmem):
            pltpu.sync_copy(x_hbm.at[i_vmem.at[0]], o_vmem)

        pltpu.emit_pipeline(
            body,
            grid=(IPAD // GW,),
            in_specs=[pl.BlockSpec((1, GW), lambda i: (0, i))],
            out_specs=[pl.BlockSpec((GW, vdim), lambda i: (i, 0))],
            core_axis_name=("core", "subcore"),
            dimension_semantics=(pltpu.PARALLEL,),
        )(i_hbm, o_hbm)

    return gather_k


@functools.lru_cache
def _sc_scatter_add():
    mesh = plsc.VectorSubcoreMesh(
        core_axis_name="core", subcore_axis_name="subcore",
        num_cores=2, num_subcores=16)
    rows_per_sub = NPAD // 16
    ncopies = rows_per_sub // GW

    nchunks = EPAD // GW // 32

    @functools.partial(
        pl.kernel,
        out_type=jax.ShapeDtypeStruct((2, NPAD, D), _f32),
        mesh=mesh,
        scratch_types=[
            pltpu.VMEM_SHARED((NPAD, D), _f32),
            pltpu.VMEM((GW, D), _f32),
            pltpu.VMEM((1, GW), jnp.int32),
        ],
    )
    def scatter_k(e_hbm, i_hbm, o_hbm, acc, ebuf, ibuf):
        cid = lax.axis_index("core")
        sid = lax.axis_index("subcore")
        wid = sid * 2 + cid

        @pl.loop(0, GW)
        def _zrow(i):
            @pl.loop(0, D, step=16)
            def _zcol(j):
                ebuf[i, pl.ds(j, 16)] = jnp.zeros((16,), _f32)

        for k in range(ncopies):
            pltpu.sync_copy(ebuf, acc.at[pl.ds(sid * rows_per_sub + k * GW, GW)])
        plsc.subcore_barrier()

        @pl.loop(0, nchunks)
        def _chunk(k):
            c = wid * nchunks + k
            pltpu.sync_copy(e_hbm.at[pl.ds(c * GW, GW)], ebuf)
            pltpu.sync_copy(i_hbm.at[pl.ds(c, 1)], ibuf)
            pltpu.sync_copy(ebuf, acc.at[ibuf.at[0]], add=True)

        plsc.subcore_barrier()

        for k in range(ncopies):
            r0 = sid * rows_per_sub + k * GW
            pltpu.sync_copy(acc.at[pl.ds(r0, GW)], o_hbm.at[cid, pl.ds(r0, GW)])

    return scatter_k


def _bdot(x, w):
    return jnp.dot(x, w, preferred_element_type=_f32)


def _ln(y, g_ref, b_ref):
    m = jnp.mean(y, axis=-1, keepdims=True)
    yc = y - m
    v = jnp.mean(yc * yc, axis=-1, keepdims=True)
    return yc / jnp.sqrt(v + 1e-5) * g_ref[...] + b_ref[...]


def _enc_node_body(x_ref, m_ref, s_ref, w1, b1, w2, b2, w3, b3, g, b, o_ref):
    x = (x_ref[...] - m_ref[...]) / (s_ref[...] + 1e-8)
    h = jax.nn.relu(_bdot(x, w1[...]) + b1[...])
    h = jax.nn.relu(_bdot(h, w2[...]) + b2[...])
    y = _bdot(h, w3[...]) + b3[...]
    o_ref[...] = _ln(y, g, b)


def _enc_edge_body(sp_ref, rp_ref, m_ref, s_ref, w1, b1, w2, b2, w3, b3, g, b, o_ref):
    d = sp_ref[...] - rp_ref[...]
    dx = d[:, 0:1]
    dy = d[:, 1:2]
    nr = jnp.sqrt(dx * dx + dy * dy)
    dxn = (dx - m_ref[0:1, 0:1]) / (s_ref[0:1, 0:1] + 1e-8)
    dyn = (dy - m_ref[0:1, 1:2]) / (s_ref[0:1, 1:2] + 1e-8)
    nrn = (nr - m_ref[0:1, 2:3]) / (s_ref[0:1, 2:3] + 1e-8)
    ein = jnp.concatenate([dxn, dyn, nrn], axis=-1)
    h = jax.nn.relu(_bdot(ein, w1[0:3, :]) + b1[...])
    h = jax.nn.relu(_bdot(h, w2[...]) + b2[...])
    y = _bdot(h, w3[...]) + b3[...]
    o_ref[...] = _ln(y, g, b)


def _edge_mlp_body(ef_ref, sv_ref, rv_ref, w1, b1, w2, b2, w3, b3, g, b,
                   enew_ref, efnew_ref):
    ef = ef_ref[...]
    x = jnp.concatenate([ef, sv_ref[...], rv_ref[...]], axis=-1)
    h = jax.nn.relu(_bdot(x, w1[...]) + b1[...])
    h = jax.nn.relu(_bdot(h, w2[...]) + b2[...])
    y = _bdot(h, w3[...]) + b3[...]
    e_new = _ln(y, g, b)
    enew_ref[...] = e_new
    efnew_ref[...] = ef + e_new


def _node_mlp_body(v_ref, a0_ref, a1_ref, w1, b1, w2, b2, w3, b3, g, b, o_ref):
    v = v_ref[...]
    agg = a0_ref[...] + a1_ref[...]
    x = jnp.concatenate([v, agg], axis=-1)
    h = jax.nn.relu(_bdot(x, w1[...]) + b1[...])
    h = jax.nn.relu(_bdot(h, w2[...]) + b2[...])
    y = _bdot(h, w3[...]) + b3[...]
    o_ref[...] = v + _ln(y, g, b)


def _decoder_body(v_ref, prev_ref, s1_ref, nt1_ref, w1, b1, w2, b2, w3, b3,
                  om_ref, osd_ref, y_ref, ns_ref, tg_ref):
    h = jax.nn.relu(_bdot(v_ref[...], w1[...]) + b1[...])
    h = jax.nn.relu(_bdot(h, w2[...]) + b2[...])
    y = _bdot(h, w3[...]) + b3[...]
    prev = prev_ref[...]
    s1 = s1_ref[...]
    om = om_ref[...]
    osd = osd_ref[...]
    ns = prev + (y * osd + om)
    nt = nt1_ref[...]
    mask = jnp.logical_or(
        jnp.logical_or(nt[:, 4:5] == 1.0, nt[:, 6:7] == 1.0), nt[:, 2:3] == 1.0)
    ns = jnp.where(mask, s1, ns)
    y_ref[...] = y
    ns_ref[...] = ns
    tg_ref[...] = (s1 - prev - om) / (osd + 1e-8)


def _full(shape):
    return pl.BlockSpec(shape, lambda i: (0, 0))


def _pad_rows(x, rows):
    return jnp.pad(x, ((0, rows - x.shape[0]), (0, 0)))


def _pad2(x, rows, cols):
    return jnp.pad(x, ((0, rows - x.shape[0]), (0, cols - x.shape[1])))


def _vec(x, cols, pad_value=0.0):
    x = x.reshape(1, -1)
    return jnp.pad(x, ((0, 0), (0, cols - x.shape[1])), constant_values=pad_value)


def kernel(mesh_pos, edges, state, node_type, parameters, params):
    state0 = state[0, 0]
    state1 = state[0, 1]
    nt0 = node_type[0, 0]
    nt1 = node_type[0, 1]
    par0 = parameters[0, 0]
    mp = mesh_pos[0, 0]
    eg = edges[0, 0].astype(jnp.int32)

    s_idx = eg[:, 0]
    r_idx = eg[:, 1]
    zpad = jnp.zeros((EPAD - E,), jnp.int32)
    idx_all = jnp.concatenate([s_idx, zpad, r_idx, zpad]).reshape(1, IPAD)
    ridx = jnp.concatenate([r_idx, jnp.full((EPAD - E,), N, jnp.int32)]).reshape(EPAD // GW, GW)

    p = params
    fv, fe, dec = p["fv"], p["fe"], p["decoder"]

    mp_pad = _pad2(mp, NPAD, D)
    gmp = _sc_gather(D)(mp_pad, idx_all)

    nm_e_m = _vec(p["nm_edges"][0], 8)
    nm_e_s = _vec(p["nm_edges"][1], 8, 1.0)
    fe_w1 = _pad2(fe["l1"][0], 8, D)

    ef = pl.pallas_call(
        _enc_edge_body,
        grid=(EPAD // BR,),
        in_specs=[
            pl.BlockSpec((BR, D), lambda i: (i, 0)),
            pl.BlockSpec((BR, D), lambda i: (i + EPAD // BR, 0)),
            _full((1, 8)), _full((1, 8)),
            _full((8, D)), _full((1, D)),
            _full((D, D)), _full((1, D)),
            _full((D, D)), _full((1, D)),
            _full((1, D)), _full((1, D)),
        ],
        out_specs=pl.BlockSpec((BR, D), lambda i: (i, 0)),
        out_shape=_sds((EPAD, D)),
    )(gmp, gmp, nm_e_m, nm_e_s,
      fe_w1, fe["l1"][1].reshape(1, D),
      fe["l2"][0], fe["l2"][1].reshape(1, D),
      fe["l3"][0], fe["l3"][1].reshape(1, D),
      fe["ln"][0].reshape(1, D), fe["ln"][1].reshape(1, D))

    vin = jnp.concatenate([state0, nt0, par0], axis=-1)
    vin_pad = _pad2(vin, NPAD, 16)
    in_dim = vin.shape[1]
    nm_n_m = _vec(p["nm_nodes"][0], 16)
    nm_n_s = _vec(p["nm_nodes"][1], 16, 1.0)
    fv_w1 = _pad2(fv["l1"][0], 16, D)

    v = pl.pallas_call(
        _enc_node_body,
        grid=(NPAD // BR,),
        in_specs=[
            pl.BlockSpec((BR, 16), lambda i: (i, 0)),
            _full((1, 16)), _full((1, 16)),
            _full((16, D)), _full((1, D)),
            _full((D, D)), _full((1, D)),
            _full((D, D)), _full((1, D)),
            _full((1, D)), _full((1, D)),
        ],
        out_specs=pl.BlockSpec((BR, D), lambda i: (i, 0)),
        out_shape=_sds((NPAD, D)),
    )(vin_pad, nm_n_m, nm_n_s,
      fv_w1, fv["l1"][1].reshape(1, D),
      fv["l2"][0], fv["l2"][1].reshape(1, D),
      fv["l3"][0], fv["l3"][1].reshape(1, D),
      fv["ln"][0].reshape(1, D), fv["ln"][1].reshape(1, D))

    edge_call = pl.pallas_call(
        _edge_mlp_body,
        grid=(EPAD // BR,),
        in_specs=[
            pl.BlockSpec((BR, D), lambda i: (i, 0)),
            pl.BlockSpec((BR, D), lambda i: (i, 0)),
            pl.BlockSpec((BR, D), lambda i: (i + EPAD // BR, 0)),
            _full((3 * D, D)), _full((1, D)),
            _full((D, D)), _full((1, D)),
            _full((D, D)), _full((1, D)),
            _full((1, D)), _full((1, D)),
        ],
        out_specs=[
            pl.BlockSpec((BR, D), lambda i: (i, 0)),
            pl.BlockSpec((BR, D), lambda i: (i, 0)),
        ],
        out_shape=[_sds((EPAD, D)), _sds((EPAD, D))],
    )

    node_call = pl.pallas_call(
        _node_mlp_body,
        grid=(NPAD // BR,),
        in_specs=[
            pl.BlockSpec((BR, D), lambda i: (i, 0)),
            pl.BlockSpec((BR, D), lambda i: (i, 0)),
            pl.BlockSpec((BR, D), lambda i: (i + NPAD // BR, 0)),
            _full((2 * D, D)), _full((1, D)),
            _full((D, D)), _full((1, D)),
            _full((D, D)), _full((1, D)),
            _full((1, D)), _full((1, D)),
        ],
        out_specs=pl.BlockSpec((BR, D), lambda i: (i, 0)),
        out_shape=_sds((NPAD, D)),
    )

    for gp in p["gnn"]:
        ew, nw = gp["edge"], gp["node"]
        gath = _sc_gather(D)(v, idx_all)
        e_new, ef = edge_call(
            ef, gath, gath,
            ew["l1"][0], ew["l1"][1].reshape(1, D),
            ew["l2"][0], ew["l2"][1].reshape(1, D),
            ew["l3"][0], ew["l3"][1].reshape(1, D),
            ew["ln"][0].reshape(1, D), ew["ln"][1].reshape(1, D))
        part = _sc_scatter_add()(e_new, ridx).reshape(2 * NPAD, D)
        v = node_call(
            v, part, part,
            nw["l1"][0], nw["l1"][1].reshape(1, D),
            nw["l2"][0], nw["l2"][1].reshape(1, D),
            nw["l3"][0], nw["l3"][1].reshape(1, D),
            nw["ln"][0].reshape(1, D), nw["ln"][1].reshape(1, D))

    prev_pad = _pad2(state0, NPAD, 8)
    s1_pad = _pad2(state1, NPAD, 8)
    nt1_pad = _pad2(nt1, NPAD, 16)
    dec_w3 = _pad2(dec["l3"][0], D, 8)
    dec_b3 = _vec(dec["l3"][1], 8)
    om = _vec(p["nm_out"][0], 8)
    osd = _vec(p["nm_out"][1], 8, 1.0)

    y, ns, tg = pl.pallas_call(
        _decoder_body,
        grid=(NPAD // BR,),
        in_specs=[
            pl.BlockSpec((BR, D), lambda i: (i, 0)),
            pl.BlockSpec((BR, 8), lambda i: (i, 0)),
            pl.BlockSpec((BR, 8), lambda i: (i, 0)),
            pl.BlockSpec((BR, 16), lambda i: (i, 0)),
            _full((D, D)), _full((1, D)),
            _full((D, D)), _full((1, D)),
            _full((D, 8)), _full((1, 8)),
            _full((1, 8)), _full((1, 8)),
        ],
        out_specs=[
            pl.BlockSpec((BR, 8), lambda i: (i, 0)),
            pl.BlockSpec((BR, 8), lambda i: (i, 0)),
            pl.BlockSpec((BR, 8), lambda i: (i, 0)),
        ],
        out_shape=[_sds((NPAD, 8)), _sds((NPAD, 8)), _sds((NPAD, 8))],
    )(v, prev_pad, s1_pad, nt1_pad,
      dec["l1"][0], dec["l1"][1].reshape(1, D),
      dec["l2"][0], dec["l2"][1].reshape(1, D),
      dec_w3, dec_b3, om, osd)

    next_output = y[:N, :3]
    next_state = ns[:N, :3]
    target = tg[:N, :3]

    state_hat = jnp.stack([state0, next_state], 0)[None]
    outputs = next_output[None, None]
    targets = target[None, None]
    return (state_hat, outputs, targets)
